# baseline (speedup 1.0000x reference)
import functools

import jax
import jax.numpy as jnp
from jax import lax
from jax.experimental import pallas as pl
from jax.experimental.pallas import tpu as pltpu
from jax.experimental.pallas import tpu_sc as plsc

_VOCAB = 1000
_DIM = 128
_BATCH = 4096
_SEQ = 200
_NCLASS = 3
_REP = 16
_TW = _NCLASS * _REP

_NC, _NS = 2, 16
_NW = _NC * _NS
_ROWS = _BATCH // _NW
_GROUPS = _ROWS // 16


def _table_body(emb_ref, w_ref, b_ref, out_ref):
    j = lax.broadcasted_iota(jnp.int32, (_TW, _NCLASS), 0)
    c = lax.broadcasted_iota(jnp.int32, (_TW, _NCLASS), 1)
    rep = ((j // _REP) == c).astype(jnp.float32)
    wrep = jnp.dot(rep, w_ref[...],
                   preferred_element_type=jnp.float32)
    brep = jnp.dot(rep, b_ref[...].T,
                   preferred_element_type=jnp.float32)
    t = lax.dot_general(emb_ref[...], wrep, (((1,), (1,)), ((), ())),
                        preferred_element_type=jnp.float32,
                        precision=lax.Precision.HIGHEST)
    out_ref[...] = (t + brep.T) * (1.0 / _SEQ)


def _make_table(emb, W, b):
    return pl.pallas_call(
        _table_body,
        out_shape=jax.ShapeDtypeStruct((_VOCAB, _TW), jnp.float32),
    )(emb, W, b.reshape(1, _NCLASS))


def _prescale_body(x_ref, out_ref):
    row = lax.broadcasted_iota(jnp.int32, x_ref.shape, 0)
    out_ref[...] = x_ref[...] * _TW + (row & (_REP - 1))


def _prescale(x):
    blk = 512
    return pl.pallas_call(
        _prescale_body,
        grid=(_BATCH // blk,),
        in_specs=[pl.BlockSpec((blk, _SEQ), lambda i: (i, 0))],
        out_specs=pl.BlockSpec((blk, _SEQ), lambda i: (i, 0)),
        out_shape=jax.ShapeDtypeStruct((_BATCH, _SEQ), jnp.int32),
    )(x)


def _sc_body(xt_hbm, tp_hbm, o_hbm, xv, tpv, ov, sem1, sem2):
    wid = lax.axis_index("s") * _NC + lax.axis_index("c")
    base = wid * _ROWS

    cp1 = pltpu.async_copy(xt_hbm.at[:, pl.ds(base, _ROWS)], xv, sem1)
    cp2 = pltpu.async_copy(tp_hbm, tpv, sem2)
    cp1.wait()
    cp2.wait()

    lane = lax.iota(jnp.int32, 16)
    t1 = tpv.at[pl.ds(_REP, _VOCAB * _TW - _REP)]
    t2 = tpv.at[pl.ds(2 * _REP, _VOCAB * _TW - 2 * _REP)]

    for g in range(_GROUPS):
        def lbody(l, accs):
            a0, a1, a2 = accs
            pos = xv[l, pl.ds(g * 16, 16)]
            a0 = a0 + plsc.load_gather(tpv, [pos])
            a1 = a1 + plsc.load_gather(t1, [pos])
            a2 = a2 + plsc.load_gather(t2, [pos])
            return (a0, a1, a2)

        z = jnp.zeros((16,), jnp.float32)
        a0, a1, a2 = plsc.parallel_loop(
            0, _SEQ, 1, unroll=4, carry=(z, z, z))(lbody)
        rowpos = (g * 16 + lane) * _NCLASS
        plsc.store_scatter(ov, [rowpos], a0)
        plsc.store_scatter(ov, [rowpos + 1], a1)
        plsc.store_scatter(ov, [rowpos + 2], a2)

    pltpu.sync_copy(ov, o_hbm.at[pl.ds(base * _NCLASS, _ROWS * _NCLASS)])


@functools.cache
def _sc_gather_reduce():
    return pl.kernel(
        _sc_body,
        out_type=jax.ShapeDtypeStruct((_BATCH * _NCLASS,), jnp.float32),
        mesh=plsc.VectorSubcoreMesh(core_axis_name="c", subcore_axis_name="s",
                                    num_cores=_NC, num_subcores=_NS),
        compiler_params=pltpu.CompilerParams(needs_layout_passes=False),
        scratch_types=(
            pltpu.VMEM((_SEQ, _ROWS), jnp.int32),
            pltpu.VMEM((_VOCAB * _TW,), jnp.float32),
            pltpu.VMEM((_ROWS * _NCLASS,), jnp.float32),
            pltpu.SemaphoreType.DMA,
            pltpu.SemaphoreType.DMA,
        ),
    )


def kernel(x, emb, W, b):
    tp = _make_table(emb, W, b).reshape(-1)
    xt = jnp.transpose(_prescale(x.astype(jnp.int32)))
    flat = _sc_gather_reduce()(xt, tp)
    return flat.reshape(_BATCH, _NCLASS)


# speedup vs baseline: 1.2810x; 1.2810x over previous
import functools

import jax
import jax.numpy as jnp
from jax import lax
from jax.experimental import pallas as pl
from jax.experimental.pallas import tpu as pltpu
from jax.experimental.pallas import tpu_sc as plsc

_VOCAB = 1000
_DIM = 128
_BATCH = 4096
_SEQ = 200
_NCLASS = 3
_REP = 16
_TW = _NCLASS * _REP

_NC, _NS = 2, 16
_NW = _NC * _NS
_ROWS = _BATCH // _NW
_GROUPS = _ROWS // 16


def _table_body(emb_ref, w_ref, b_ref, out_ref):
    j = lax.broadcasted_iota(jnp.int32, (_TW, _NCLASS), 0)
    c = lax.broadcasted_iota(jnp.int32, (_TW, _NCLASS), 1)
    rep = ((j // _REP) == c).astype(jnp.float32)
    wrep = jnp.dot(rep, w_ref[...],
                   preferred_element_type=jnp.float32)
    brep = jnp.dot(rep, b_ref[...].T,
                   preferred_element_type=jnp.float32)
    t = lax.dot_general(emb_ref[...], wrep, (((1,), (1,)), ((), ())),
                        preferred_element_type=jnp.float32,
                        precision=lax.Precision.HIGHEST)
    out_ref[...] = (t + brep.T) * (1.0 / _SEQ)


def _make_table(emb, W, b):
    return pl.pallas_call(
        _table_body,
        out_shape=jax.ShapeDtypeStruct((_VOCAB, _TW), jnp.float32),
    )(emb, W, b.reshape(1, _NCLASS))


def _prescale_body(xt_ref, out_ref):
    col = lax.broadcasted_iota(jnp.int32, xt_ref.shape, 1)
    out_ref[...] = xt_ref[...] * _TW + (col & (_REP - 1))


def _prescale(xt):
    blk = 40
    return pl.pallas_call(
        _prescale_body,
        grid=(_SEQ // blk,),
        in_specs=[pl.BlockSpec((blk, _BATCH), lambda i: (i, 0))],
        out_specs=pl.BlockSpec((blk, _BATCH), lambda i: (i, 0)),
        out_shape=jax.ShapeDtypeStruct((_SEQ, _BATCH), jnp.int32),
    )(xt)


def _sc_body(xt_hbm, tp_hbm, o_hbm, xv, tpv, ov, sem1, sem2):
    wid = lax.axis_index("s") * _NC + lax.axis_index("c")
    base = wid * _ROWS

    cp1 = pltpu.async_copy(xt_hbm.at[:, pl.ds(base, _ROWS)], xv, sem1)
    cp2 = pltpu.async_copy(tp_hbm, tpv, sem2)
    cp1.wait()
    cp2.wait()

    lane = lax.iota(jnp.int32, 16)
    t1 = tpv.at[pl.ds(_REP, _VOCAB * _TW - _REP)]
    t2 = tpv.at[pl.ds(2 * _REP, _VOCAB * _TW - 2 * _REP)]

    for g in range(_GROUPS):
        def lbody(l, accs):
            a0, a1, a2 = accs
            pos = xv[l, pl.ds(g * 16, 16)]
            a0 = a0 + plsc.load_gather(tpv, [pos])
            a1 = a1 + plsc.load_gather(t1, [pos])
            a2 = a2 + plsc.load_gather(t2, [pos])
            return (a0, a1, a2)

        z = jnp.zeros((16,), jnp.float32)
        a0, a1, a2 = plsc.parallel_loop(
            0, _SEQ, 1, unroll=4, carry=(z, z, z))(lbody)
        rowpos = (g * 16 + lane) * _NCLASS
        plsc.store_scatter(ov, [rowpos], a0)
        plsc.store_scatter(ov, [rowpos + 1], a1)
        plsc.store_scatter(ov, [rowpos + 2], a2)

    pltpu.sync_copy(ov, o_hbm.at[pl.ds(base * _NCLASS, _ROWS * _NCLASS)])


@functools.cache
def _sc_gather_reduce():
    return pl.kernel(
        _sc_body,
        out_type=jax.ShapeDtypeStruct((_BATCH * _NCLASS,), jnp.float32),
        mesh=plsc.VectorSubcoreMesh(core_axis_name="c", subcore_axis_name="s",
                                    num_cores=_NC, num_subcores=_NS),
        compiler_params=pltpu.CompilerParams(needs_layout_passes=False),
        scratch_types=(
            pltpu.VMEM((_SEQ, _ROWS), jnp.int32),
            pltpu.VMEM((_VOCAB * _TW,), jnp.float32),
            pltpu.VMEM((_ROWS * _NCLASS,), jnp.float32),
            pltpu.SemaphoreType.DMA,
            pltpu.SemaphoreType.DMA,
        ),
    )


def kernel(x, emb, W, b):
    tp = _make_table(emb, W, b).reshape(-1)
    xt = _prescale(jnp.transpose(x.astype(jnp.int32)))
    flat = _sc_gather_reduce()(xt, tp)
    return flat.reshape(_BATCH, _NCLASS)


# speedup vs baseline: 1.4455x; 1.1284x over previous
import functools

import jax
import jax.numpy as jnp
from jax import lax
from jax.experimental import pallas as pl
from jax.experimental.pallas import tpu as pltpu
from jax.experimental.pallas import tpu_sc as plsc

_VOCAB = 1000
_DIM = 128
_BATCH = 4096
_SEQ = 200
_NCLASS = 3
_REP = 16
_TW = _NCLASS * _REP

_NC, _NS = 2, 16
_NW = _NC * _NS
_ROWS = _BATCH // _NW
_GROUPS = _ROWS // 16


def _table_body(emb_ref, w_ref, b_ref, out_ref):
    j = lax.broadcasted_iota(jnp.int32, (_TW, _NCLASS), 0)
    c = lax.broadcasted_iota(jnp.int32, (_TW, _NCLASS), 1)
    rep = ((j // _REP) == c).astype(jnp.float32)
    wrep = jnp.dot(rep, w_ref[...],
                   preferred_element_type=jnp.float32)
    brep = jnp.dot(rep, b_ref[...].T,
                   preferred_element_type=jnp.float32)
    t = lax.dot_general(emb_ref[...], wrep, (((1,), (1,)), ((), ())),
                        preferred_element_type=jnp.float32,
                        precision=lax.Precision.HIGHEST)
    out_ref[...] = (t + brep.T) * (1.0 / _SEQ)


def _make_table(emb, W, b):
    return pl.pallas_call(
        _table_body,
        out_shape=jax.ShapeDtypeStruct((_VOCAB, _TW), jnp.float32),
    )(emb, W, b.reshape(1, _NCLASS))


def _sc_body(xt_hbm, tp_hbm, o_hbm, xv, tpv, ov, sem1, sem2):
    wid = lax.axis_index("s") * _NC + lax.axis_index("c")
    base = wid * _ROWS

    cp1 = pltpu.async_copy(xt_hbm.at[:, pl.ds(base, _ROWS)], xv, sem1)
    cp2 = pltpu.async_copy(tp_hbm, tpv, sem2)
    cp1.wait()
    cp2.wait()

    lane = lax.iota(jnp.int32, 16)
    t1 = tpv.at[pl.ds(_REP, _VOCAB * _TW - _REP)]
    t2 = tpv.at[pl.ds(2 * _REP, _VOCAB * _TW - 2 * _REP)]

    for g in range(_GROUPS):
        def lbody(l, accs):
            a0, a1, a2 = accs
            pos = xv[l, pl.ds(g * 16, 16)] * _TW + lane
            a0 = a0 + plsc.load_gather(tpv, [pos])
            a1 = a1 + plsc.load_gather(t1, [pos])
            a2 = a2 + plsc.load_gather(t2, [pos])
            return (a0, a1, a2)

        z = jnp.zeros((16,), jnp.float32)
        a0, a1, a2 = plsc.parallel_loop(
            0, _SEQ, 1, unroll=8, carry=(z, z, z))(lbody)
        rowpos = (g * 16 + lane) * _NCLASS
        plsc.store_scatter(ov, [rowpos], a0)
        plsc.store_scatter(ov, [rowpos + 1], a1)
        plsc.store_scatter(ov, [rowpos + 2], a2)

    pltpu.sync_copy(ov, o_hbm.at[pl.ds(base * _NCLASS, _ROWS * _NCLASS)])


@functools.cache
def _sc_gather_reduce():
    return pl.kernel(
        _sc_body,
        out_type=jax.ShapeDtypeStruct((_BATCH * _NCLASS,), jnp.float32),
        mesh=plsc.VectorSubcoreMesh(core_axis_name="c", subcore_axis_name="s",
                                    num_cores=_NC, num_subcores=_NS),
        compiler_params=pltpu.CompilerParams(needs_layout_passes=False),
        scratch_types=(
            pltpu.VMEM((_SEQ, _ROWS), jnp.int32),
            pltpu.VMEM((_VOCAB * _TW,), jnp.float32),
            pltpu.VMEM((_ROWS * _NCLASS,), jnp.float32),
            pltpu.SemaphoreType.DMA,
            pltpu.SemaphoreType.DMA,
        ),
    )


def kernel(x, emb, W, b):
    tp = _make_table(emb, W, b).reshape(-1)
    xt = jnp.transpose(x.astype(jnp.int32))
    flat = _sc_gather_reduce()(xt, tp)
    return flat.reshape(_BATCH, _NCLASS)


# speedup vs baseline: 1.5623x; 1.0808x over previous
import functools

import jax
import jax.numpy as jnp
from jax import lax
from jax.experimental import pallas as pl
from jax.experimental.pallas import tpu as pltpu
from jax.experimental.pallas import tpu_sc as plsc

_VOCAB = 1000
_DIM = 128
_BATCH = 4096
_SEQ = 200
_NCLASS = 3
_REP = 16
_TW = _NCLASS * _REP

_NC, _NS = 2, 16
_NW = _NC * _NS
_ROWS = _BATCH // _NW
_GROUPS = _ROWS // 16


def _table_body(emb_ref, w_ref, b_ref, out_ref):
    j = lax.broadcasted_iota(jnp.int32, (_TW, _NCLASS), 0)
    c = lax.broadcasted_iota(jnp.int32, (_TW, _NCLASS), 1)
    rep = ((j // _REP) == c).astype(jnp.float32)
    wrep = jnp.dot(rep, w_ref[...],
                   preferred_element_type=jnp.float32)
    brep = jnp.dot(rep, b_ref[...].T,
                   preferred_element_type=jnp.float32)
    t = lax.dot_general(emb_ref[...], wrep, (((1,), (1,)), ((), ())),
                        preferred_element_type=jnp.float32,
                        precision=lax.Precision.HIGHEST)
    out_ref[...] = (t + brep.T) * (1.0 / _SEQ)


def _make_table(emb, W, b):
    return pl.pallas_call(
        _table_body,
        out_shape=jax.ShapeDtypeStruct((_VOCAB, _TW), jnp.float32),
    )(emb, W, b.reshape(1, _NCLASS))


def _sc_body(xt_hbm, tp_hbm, o0_hbm, o1_hbm, o2_hbm,
             xv, tpv, o0v, o1v, o2v, sem1, sem2):
    wid = lax.axis_index("s") * _NC + lax.axis_index("c")
    base = wid * _ROWS

    cp1 = pltpu.async_copy(xt_hbm.at[:, pl.ds(base, _ROWS)], xv, sem1)
    cp2 = pltpu.async_copy(tp_hbm, tpv, sem2)
    cp1.wait()
    cp2.wait()

    lane = lax.iota(jnp.int32, 16)
    t1 = tpv.at[pl.ds(_REP, _VOCAB * _TW - _REP)]
    t2 = tpv.at[pl.ds(2 * _REP, _VOCAB * _TW - 2 * _REP)]

    for g in range(_GROUPS):
        def lbody(l, accs):
            a0, a1, a2 = accs
            pos = xv[l, pl.ds(g * 16, 16)] * _TW + lane
            a0 = a0 + plsc.load_gather(tpv, [pos])
            a1 = a1 + plsc.load_gather(t1, [pos])
            a2 = a2 + plsc.load_gather(t2, [pos])
            return (a0, a1, a2)

        z = jnp.zeros((16,), jnp.float32)
        a0, a1, a2 = plsc.parallel_loop(
            0, _SEQ, 1, unroll=4, carry=(z, z, z))(lbody)
        o0v[pl.ds(g * 16, 16)] = a0
        o1v[pl.ds(g * 16, 16)] = a1
        o2v[pl.ds(g * 16, 16)] = a2

    pltpu.sync_copy(o0v, o0_hbm.at[pl.ds(base, _ROWS)])
    pltpu.sync_copy(o1v, o1_hbm.at[pl.ds(base, _ROWS)])
    pltpu.sync_copy(o2v, o2_hbm.at[pl.ds(base, _ROWS)])


@functools.cache
def _sc_gather_reduce():
    col = jax.ShapeDtypeStruct((_BATCH,), jnp.float32)
    return pl.kernel(
        _sc_body,
        out_type=(col, col, col),
        mesh=plsc.VectorSubcoreMesh(core_axis_name="c", subcore_axis_name="s",
                                    num_cores=_NC, num_subcores=_NS),
        compiler_params=pltpu.CompilerParams(needs_layout_passes=False),
        scratch_types=(
            pltpu.VMEM((_SEQ, _ROWS), jnp.int32),
            pltpu.VMEM((_VOCAB * _TW,), jnp.float32),
            pltpu.VMEM((_ROWS,), jnp.float32),
            pltpu.VMEM((_ROWS,), jnp.float32),
            pltpu.VMEM((_ROWS,), jnp.float32),
            pltpu.SemaphoreType.DMA,
            pltpu.SemaphoreType.DMA,
        ),
    )


def kernel(x, emb, W, b):
    tp = _make_table(emb, W, b).reshape(-1)
    xt = jnp.transpose(x.astype(jnp.int32))
    o0, o1, o2 = _sc_gather_reduce()(xt, tp)
    return jnp.stack([o0, o1, o2], axis=1)
